# baseline (speedup 1.0000x reference)
import functools

import jax
import jax.numpy as jnp
from jax import lax
from jax.experimental import pallas as pl
from jax.experimental.pallas import tpu as pltpu
from jax.experimental.pallas import tpu_sc as plsc
from jax.experimental import layout as jex_layout

BATCH = 16384
EMB = 64
K = 128

NC = 2
NS = 16
NW = NC * NS
BPW = BATCH // NW
CH = 128
NCH = BPW // CH
L = 16


def _extract_chunk(idx, gbuf, mini, j):
    def ebody(v, carry):
        rv = idx[j, pl.ds(v * L, L)]
        hv = lax.bitwise_and(rv, 1) * EMB
        for k in range(L):
            off = hv[k]
            row = v * L + k
            for c in range(EMB // L):
                mini[row, pl.ds(c * L, L)] = gbuf[row, pl.ds(off + c * L, L)]
        return carry
    lax.fori_loop(0, CH // L, ebody, 0)


def _sc_gather_body(uid_hbm, iid_hbm, mfu_hbm, mfi_hbm, mlpu_hbm, mlpi_hbm,
                    o_mfu, o_mfi, o_u, o_i,
                    idx_u, idx_i, pidx_u, pidx_i, gbuf_a, gbuf_b, mini, sem):
    wid = lax.axis_index("s") * NC + lax.axis_index("c")
    pltpu.sync_copy(uid_hbm.at[wid], idx_u)
    pltpu.sync_copy(iid_hbm.at[wid], idx_i)

    for j in range(NCH):
        for v in range(CH // L):
            sl = pl.ds(v * L, L)
            pidx_u[j, sl] = lax.shift_right_logical(idx_u[j, sl], 1)
            pidx_i[j, sl] = lax.shift_right_logical(idx_i[j, sl], 1)

    for tab, pidx, idx, out in (
        (mfu_hbm, pidx_u, idx_u, o_mfu),
        (mfi_hbm, pidx_i, idx_i, o_mfi),
        (mlpu_hbm, pidx_u, idx_u, o_u),
        (mlpi_hbm, pidx_i, idx_i, o_i),
    ):
        bufs = (gbuf_a, gbuf_b)
        handles = {}
        for j in range(min(2, NCH)):
            handles[j] = pltpu.async_copy(tab.at[pidx.at[j]], bufs[j % 2], sem)
        for j in range(NCH):
            handles[j].wait()
            _extract_chunk(idx, bufs[j % 2], mini, j)
            pltpu.sync_copy(mini, out.at[wid, pl.ds(j * CH, CH)])
            if j + 2 < NCH:
                handles[j + 2] = pltpu.async_copy(
                    tab.at[pidx.at[j + 2]], bufs[j % 2], sem)


_sc_gather = functools.partial(
    pl.kernel,
    mesh=plsc.VectorSubcoreMesh(core_axis_name="c", subcore_axis_name="s"),
    out_type=tuple(
        jax.ShapeDtypeStruct((NW, BPW, EMB), jnp.float32) for _ in range(4)),
    scratch_types=[
        pltpu.VMEM((NCH, CH), jnp.int32),
        pltpu.VMEM((NCH, CH), jnp.int32),
        pltpu.VMEM((NCH, CH), jnp.int32),
        pltpu.VMEM((NCH, CH), jnp.int32),
        pltpu.VMEM((CH, 2 * EMB), jnp.float32),
        pltpu.VMEM((CH, 2 * EMB), jnp.float32),
        pltpu.VMEM((CH, EMB), jnp.float32),
        pltpu.SemaphoreType.DMA,
    ],
)(_sc_gather_body)


def _tc_mlp_body(mfu_ref, mfi_ref, xu_ref, xi_ref, w1a_ref, w1b_ref, b1_ref,
                 w2_ref, b2_ref, wa_ref, wb_ref, bout_ref, out_ref):
    f32 = jnp.float32
    h = jnp.dot(xu_ref[...], w1a_ref[...], preferred_element_type=f32)
    h += jnp.dot(xi_ref[...], w1b_ref[...], preferred_element_type=f32)
    h = jnp.maximum(h + b1_ref[...], 0.0)
    h = jnp.dot(h, w2_ref[...], preferred_element_type=f32)
    h = jnp.maximum(h + b2_ref[...], 0.0)
    xmf = mfu_ref[...] * mfi_ref[...]
    z = jnp.dot(xmf, wa_ref[...], preferred_element_type=f32)
    z += jnp.dot(h, wb_ref[...], preferred_element_type=f32)
    z += bout_ref[...]
    out_ref[...] = jnp.where(z > 0.0, z, jnp.exp(z) - 1.0)


def kernel(user_id, item_id, mf_user, mf_item, mlp_user, mlp_item,
           W1, b1, W2, b2, Wout, bout):
    uid = user_id.astype(jnp.int32).reshape(NW, NCH, CH)
    iid = item_id.astype(jnp.int32).reshape(NW, NCH, CH)
    pair = lambda t, n: t[:n].reshape(n // 2, 2 * EMB)
    mfu, mfi, xu, xi = _sc_gather(
        uid, iid,
        pair(mf_user, 1000000), pair(mf_item, 100000),
        pair(mlp_user, 1000000), pair(mlp_item, 100000))
    mfu = mfu.reshape(BATCH, EMB)
    mfi = mfi.reshape(BATCH, EMB)
    xu = xu.reshape(BATCH, EMB)
    xi = xi.reshape(BATCH, EMB)

    BLK = 2048
    grid = (BATCH // BLK,)
    zero = lambda i: (0, 0)
    out = pl.pallas_call(
        _tc_mlp_body,
        grid=grid,
        in_specs=[
            pl.BlockSpec((BLK, EMB), lambda i: (i, 0)),
            pl.BlockSpec((BLK, EMB), lambda i: (i, 0)),
            pl.BlockSpec((BLK, EMB), lambda i: (i, 0)),
            pl.BlockSpec((BLK, EMB), lambda i: (i, 0)),
            pl.BlockSpec((EMB, K), zero),
            pl.BlockSpec((EMB, K), zero),
            pl.BlockSpec((1, K), zero),
            pl.BlockSpec((K, K), zero),
            pl.BlockSpec((1, K), zero),
            pl.BlockSpec((EMB, 1), zero),
            pl.BlockSpec((K, 1), zero),
            pl.BlockSpec((1, 1), zero),
        ],
        out_specs=pl.BlockSpec((BLK, 1), lambda i: (i, 0)),
        out_shape=jax.ShapeDtypeStruct((BATCH, 1), jnp.float32),
    )(
        mfu, mfi, xu, xi,
        W1[:EMB, :], W1[EMB:, :], b1.reshape(1, K),
        W2, b2.reshape(1, K),
        Wout[:EMB, :], Wout[EMB:, :], bout.reshape(1, 1),
    )
    return out


# speedup vs baseline: 1.0219x; 1.0219x over previous
import functools

import jax
import jax.numpy as jnp
from jax import lax
from jax.experimental import pallas as pl
from jax.experimental.pallas import tpu as pltpu
from jax.experimental.pallas import tpu_sc as plsc
from jax.experimental import layout as jex_layout

BATCH = 16384
EMB = 64
K = 128

NC = 2
NS = 16
NW = NC * NS
BPW = BATCH // NW
CH = 128
NCH = BPW // CH
L = 16


def _extract_chunk(idx, gbuf, mini, j):
    def ebody(v, carry):
        rv = idx[j, pl.ds(v * L, L)]
        hv = lax.bitwise_and(rv, 1) * EMB
        for k in range(L):
            off = hv[k]
            row = v * L + k
            for c in range(EMB // L):
                mini[row, pl.ds(c * L, L)] = gbuf[row, pl.ds(off + c * L, L)]
        return carry
    lax.fori_loop(0, CH // L, ebody, 0)


def _sc_gather1_body(pid_hbm, id_hbm, tab_hbm, out,
                     idx, pidx, gbuf_a, gbuf_b, mini, sem):
    wid = lax.axis_index("s") * NC + lax.axis_index("c")
    pltpu.sync_copy(id_hbm.at[wid], idx)
    pltpu.sync_copy(pid_hbm.at[wid], pidx)

    bufs = (gbuf_a, gbuf_b)
    handles = {}
    for j in range(min(2, NCH)):
        handles[j] = pltpu.async_copy(tab_hbm.at[pidx.at[j]], bufs[j % 2], sem)
    for j in range(NCH):
        handles[j].wait()
        _extract_chunk(idx, bufs[j % 2], mini, j)
        pltpu.sync_copy(mini, out.at[wid, pl.ds(j * CH, CH)])
        if j + 2 < NCH:
            handles[j + 2] = pltpu.async_copy(
                tab_hbm.at[pidx.at[j + 2]], bufs[j % 2], sem)


_sc_gather1 = functools.partial(
    pl.kernel,
    mesh=plsc.VectorSubcoreMesh(core_axis_name="c", subcore_axis_name="s"),
    out_type=jax.ShapeDtypeStruct((NW, BPW, EMB), jnp.float32),
    scratch_types=[
        pltpu.VMEM((NCH, CH), jnp.int32),
        pltpu.VMEM((NCH, CH), jnp.int32),
        pltpu.VMEM((CH, 2 * EMB), jnp.float32),
        pltpu.VMEM((CH, 2 * EMB), jnp.float32),
        pltpu.VMEM((CH, EMB), jnp.float32),
        pltpu.SemaphoreType.DMA,
    ],
)(_sc_gather1_body)


def _tc_mlp_body(mfu_ref, mfi_ref, xu_ref, xi_ref, w1a_ref, w1b_ref, b1_ref,
                 w2_ref, b2_ref, wa_ref, wb_ref, bout_ref, out_ref):
    f32 = jnp.float32
    h = jnp.dot(xu_ref[...], w1a_ref[...], preferred_element_type=f32)
    h += jnp.dot(xi_ref[...], w1b_ref[...], preferred_element_type=f32)
    h = jnp.maximum(h + b1_ref[...], 0.0)
    h = jnp.dot(h, w2_ref[...], preferred_element_type=f32)
    h = jnp.maximum(h + b2_ref[...], 0.0)
    xmf = mfu_ref[...] * mfi_ref[...]
    z = jnp.dot(xmf, wa_ref[...], preferred_element_type=f32)
    z += jnp.dot(h, wb_ref[...], preferred_element_type=f32)
    z += bout_ref[...]
    out_ref[...] = jnp.where(z > 0.0, z, jnp.exp(z) - 1.0)


def kernel(user_id, item_id, mf_user, mf_item, mlp_user, mlp_item,
           W1, b1, W2, b2, Wout, bout):
    uid = user_id.astype(jnp.int32)
    iid = item_id.astype(jnp.int32)
    shape3 = (NW, NCH, CH)
    uid3 = uid.reshape(shape3)
    iid3 = iid.reshape(shape3)
    puid3 = (uid >> 1).reshape(shape3)
    piid3 = (iid >> 1).reshape(shape3)
    pair = lambda t, n: t[:n].reshape(n // 2, 2 * EMB)
    mfu = _sc_gather1(puid3, uid3, pair(mf_user, 1000000))
    mfi = _sc_gather1(piid3, iid3, pair(mf_item, 100000))
    xu = _sc_gather1(puid3, uid3, pair(mlp_user, 1000000))
    xi = _sc_gather1(piid3, iid3, pair(mlp_item, 100000))
    mfu = mfu.reshape(BATCH, EMB)
    mfi = mfi.reshape(BATCH, EMB)
    xu = xu.reshape(BATCH, EMB)
    xi = xi.reshape(BATCH, EMB)

    BLK = 2048
    grid = (BATCH // BLK,)
    zero = lambda i: (0, 0)
    out = pl.pallas_call(
        _tc_mlp_body,
        grid=grid,
        in_specs=[
            pl.BlockSpec((BLK, EMB), lambda i: (i, 0)),
            pl.BlockSpec((BLK, EMB), lambda i: (i, 0)),
            pl.BlockSpec((BLK, EMB), lambda i: (i, 0)),
            pl.BlockSpec((BLK, EMB), lambda i: (i, 0)),
            pl.BlockSpec((EMB, K), zero),
            pl.BlockSpec((EMB, K), zero),
            pl.BlockSpec((1, K), zero),
            pl.BlockSpec((K, K), zero),
            pl.BlockSpec((1, K), zero),
            pl.BlockSpec((EMB, 1), zero),
            pl.BlockSpec((K, 1), zero),
            pl.BlockSpec((1, 1), zero),
        ],
        out_specs=pl.BlockSpec((BLK, 1), lambda i: (i, 0)),
        out_shape=jax.ShapeDtypeStruct((BATCH, 1), jnp.float32),
    )(
        mfu, mfi, xu, xi,
        W1[:EMB, :], W1[EMB:, :], b1.reshape(1, K),
        W2, b2.reshape(1, K),
        Wout[:EMB, :], Wout[EMB:, :], bout.reshape(1, 1),
    )
    return out


# speedup vs baseline: 1.4286x; 1.3979x over previous
import functools

import jax
import jax.numpy as jnp
from jax import lax
from jax.experimental import pallas as pl
from jax.experimental.pallas import tpu as pltpu
from jax.experimental.pallas import tpu_sc as plsc
from jax.experimental import layout as jex_layout

BATCH = 16384
EMB = 64
K = 128

NC = 2
NS = 16
NW = NC * NS
BPW = BATCH // NW
CH = 128
NCH = BPW // CH
L = 16
GC = 32
NCK = BPW // GC


def _issue_chunk(tab, idx, gbuf, j, off, sem):
    handles = []
    for v in range(GC // L):
        rv = idx[j, pl.ds(off + v * L, L)]
        gv = lax.bitwise_and(rv, jnp.int32(-8))
        for k in range(L):
            base = pl.multiple_of(gv[k], 8)
            handles.append(pltpu.async_copy(
                tab.at[pl.ds(base, 8)], gbuf.at[v * L + k], sem))
    return handles


def _extract_rows(idx, gbuf, mini, j, off):
    for v in range(GC // L):
        rv = idx[j, pl.ds(off + v * L, L)]
        r8v = lax.rem(rv, 8)
        for k in range(L):
            r8 = r8v[k]
            row = v * L + k
            for c in range(EMB // L):
                sl = pl.ds(c * L, L)
                mini[row, sl] = gbuf[row, r8, sl]


def _sc_gather1_body(id_hbm, tab_hbm, out, idx, gbuf_a, gbuf_b, mini, sem):
    wid = lax.axis_index("s") * NC + lax.axis_index("c")
    pltpu.sync_copy(id_hbm.at[wid], idx)

    def pair(p, carry):
        ca = 2 * p
        cb = 2 * p + 1
        ja = lax.div(ca, NCK // NCH)
        oa = lax.rem(ca, NCK // NCH) * GC
        jb = lax.div(cb, NCK // NCH)
        ob = lax.rem(cb, NCK // NCH) * GC
        ha = _issue_chunk(tab_hbm, idx, gbuf_a, ja, oa, sem)
        hb = _issue_chunk(tab_hbm, idx, gbuf_b, jb, ob, sem)
        for h_ in ha:
            h_.wait()
        _extract_rows(idx, gbuf_a, mini, ja, oa)
        pltpu.sync_copy(
            mini, out.at[wid, pl.ds(pl.multiple_of(ca * GC, GC), GC)])
        for h_ in hb:
            h_.wait()
        _extract_rows(idx, gbuf_b, mini, jb, ob)
        pltpu.sync_copy(
            mini, out.at[wid, pl.ds(pl.multiple_of(cb * GC, GC), GC)])
        return carry
    lax.fori_loop(0, NCK // 2, pair, 0)


_sc_gather1 = functools.partial(
    pl.kernel,
    mesh=plsc.VectorSubcoreMesh(core_axis_name="c", subcore_axis_name="s"),
    out_type=jax.ShapeDtypeStruct((NW, BPW, EMB), jnp.float32),
    scratch_types=[
        pltpu.VMEM((NCH, CH), jnp.int32),
        pltpu.VMEM((GC, 8, EMB), jnp.float32),
        pltpu.VMEM((GC, 8, EMB), jnp.float32),
        pltpu.VMEM((GC, EMB), jnp.float32),
        pltpu.SemaphoreType.DMA,
    ],
)(_sc_gather1_body)


def _tc_mlp_body(mfu_ref, mfi_ref, xu_ref, xi_ref, w1a_ref, w1b_ref, b1_ref,
                 w2_ref, b2_ref, wa_ref, wb_ref, bout_ref, out_ref):
    f32 = jnp.float32
    h = jnp.dot(xu_ref[...], w1a_ref[...], preferred_element_type=f32)
    h += jnp.dot(xi_ref[...], w1b_ref[...], preferred_element_type=f32)
    h = jnp.maximum(h + b1_ref[...], 0.0)
    h = jnp.dot(h, w2_ref[...], preferred_element_type=f32)
    h = jnp.maximum(h + b2_ref[...], 0.0)
    xmf = mfu_ref[...] * mfi_ref[...]
    z = jnp.dot(xmf, wa_ref[...], preferred_element_type=f32)
    z += jnp.dot(h, wb_ref[...], preferred_element_type=f32)
    z += bout_ref[...]
    out_ref[...] = jnp.where(z > 0.0, z, jnp.exp(z) - 1.0)


def kernel(user_id, item_id, mf_user, mf_item, mlp_user, mlp_item,
           W1, b1, W2, b2, Wout, bout):
    uid = user_id.astype(jnp.int32)
    iid = item_id.astype(jnp.int32)
    shape3 = (NW, NCH, CH)
    uid3 = uid.reshape(shape3)
    iid3 = iid.reshape(shape3)
    mfu = _sc_gather1(uid3, mf_user[:1000000])
    mfi = _sc_gather1(iid3, mf_item[:100000])
    xu = _sc_gather1(uid3, mlp_user[:1000000])
    xi = _sc_gather1(iid3, mlp_item[:100000])
    mfu = mfu.reshape(BATCH, EMB)
    mfi = mfi.reshape(BATCH, EMB)
    xu = xu.reshape(BATCH, EMB)
    xi = xi.reshape(BATCH, EMB)

    BLK = 2048
    grid = (BATCH // BLK,)
    zero = lambda i: (0, 0)
    out = pl.pallas_call(
        _tc_mlp_body,
        grid=grid,
        in_specs=[
            pl.BlockSpec((BLK, EMB), lambda i: (i, 0)),
            pl.BlockSpec((BLK, EMB), lambda i: (i, 0)),
            pl.BlockSpec((BLK, EMB), lambda i: (i, 0)),
            pl.BlockSpec((BLK, EMB), lambda i: (i, 0)),
            pl.BlockSpec((EMB, K), zero),
            pl.BlockSpec((EMB, K), zero),
            pl.BlockSpec((1, K), zero),
            pl.BlockSpec((K, K), zero),
            pl.BlockSpec((1, K), zero),
            pl.BlockSpec((EMB, 1), zero),
            pl.BlockSpec((K, 1), zero),
            pl.BlockSpec((1, 1), zero),
        ],
        out_specs=pl.BlockSpec((BLK, 1), lambda i: (i, 0)),
        out_shape=jax.ShapeDtypeStruct((BATCH, 1), jnp.float32),
    )(
        mfu, mfi, xu, xi,
        W1[:EMB, :], W1[EMB:, :], b1.reshape(1, K),
        W2, b2.reshape(1, K),
        Wout[:EMB, :], Wout[EMB:, :], bout.reshape(1, 1),
    )
    return out
